# baseline (speedup 1.0000x reference)
import jax
import jax.numpy as jnp
from jax import lax
from jax.experimental import pallas as pl
from jax.experimental.pallas import tpu as pltpu

_NUM_EMB = 1024
_DIM = 64
_TOP_K = 3
_COMMIT = 0.25


def _vq_body(x_ref, e_ref, dist_ref, idx_ref, q_ref, loss_ref, perp_ref,
             cnt_ref, sse_ref, *, nb, n_total, num_blocks):
    i = pl.program_id(0)
    x = x_ref[...]
    e = e_ref[...]
    x2 = jnp.sum(x * x, axis=1, keepdims=True)
    e2 = jnp.sum(e * e, axis=1, keepdims=True).T
    xe = jnp.dot(x, e.T, preferred_element_type=jnp.float32)
    d = x2 + e2 - 2.0 * xe
    dist_ref[...] = d

    iota = lax.broadcasted_iota(jnp.int32, (nb, _NUM_EMB), 1)
    work = d
    cnt = jnp.zeros((1, _NUM_EMB), jnp.float32)
    sse = jnp.zeros((), jnp.float32)
    idx_cols = []
    q_slabs = []
    for _ in range(_TOP_K):
        idxj = jnp.argmin(work, axis=1).astype(jnp.int32)[:, None]
        hit = iota == idxj
        oh = hit.astype(jnp.float32)
        qj = jnp.dot(oh, e, preferred_element_type=jnp.float32)
        idx_cols.append(idxj)
        q_slabs.append(qj[:, None, :])
        cnt = cnt + jnp.sum(oh, axis=0, keepdims=True)
        diff = qj - x
        sse = sse + jnp.sum(diff * diff)
        work = jnp.where(hit, jnp.inf, work)

    idx_ref[...] = jnp.concatenate(idx_cols, axis=1)
    q_ref[...] = jnp.concatenate(q_slabs, axis=1)

    sse11 = sse.reshape(1, 1)

    @pl.when(i == 0)
    def _init():
        cnt_ref[...] = cnt
        sse_ref[...] = sse11
        loss_ref[...] = jnp.zeros((1, 1), jnp.float32)
        perp_ref[...] = jnp.zeros((1, 1), jnp.float32)

    @pl.when(i > 0)
    def _acc():
        cnt_ref[...] = cnt_ref[...] + cnt
        sse_ref[...] = sse_ref[...] + sse11

    @pl.when(i == num_blocks - 1)
    def _fin():
        avg = cnt_ref[...] / n_total
        ent = jnp.sum(avg * jnp.log(avg + 1e-10), keepdims=True)
        perp_ref[...] = jnp.exp(-ent).reshape(1, 1)
        loss_ref[...] = sse_ref[...] * (_COMMIT / (n_total * _TOP_K * _DIM))


def kernel(inputs, embedding_weight):
    B, T, C = inputs.shape
    N = B * T
    K = embedding_weight.shape[0]
    nb = 512
    num_blocks = N // nb
    flat = inputs.reshape(N, C)

    import functools
    dist, idx, q, loss, perp = pl.pallas_call(
        functools.partial(_vq_body, nb=nb, n_total=float(N),
                          num_blocks=num_blocks),
        grid=(num_blocks,),
        in_specs=[
            pl.BlockSpec((nb, C), lambda i: (i, 0)),
            pl.BlockSpec((K, C), lambda i: (0, 0)),
        ],
        out_specs=[
            pl.BlockSpec((nb, K), lambda i: (i, 0)),
            pl.BlockSpec((nb, _TOP_K), lambda i: (i, 0)),
            pl.BlockSpec((nb, _TOP_K, C), lambda i: (i, 0, 0)),
            pl.BlockSpec((1, 1), lambda i: (0, 0)),
            pl.BlockSpec((1, 1), lambda i: (0, 0)),
        ],
        out_shape=[
            jax.ShapeDtypeStruct((N, K), jnp.float32),
            jax.ShapeDtypeStruct((N, _TOP_K), jnp.int32),
            jax.ShapeDtypeStruct((N, _TOP_K, C), jnp.float32),
            jax.ShapeDtypeStruct((1, 1), jnp.float32),
            jax.ShapeDtypeStruct((1, 1), jnp.float32),
        ],
        scratch_shapes=[
            pltpu.VMEM((1, K), jnp.float32),
            pltpu.VMEM((1, 1), jnp.float32),
        ],
    )(flat, embedding_weight)

    quantized_st = q.reshape(B, T, _TOP_K, C)
    return (loss[0, 0], quantized_st, perp[0, 0], idx, dist)


# speedup vs baseline: 1.2316x; 1.2316x over previous
import jax
import jax.numpy as jnp
from jax import lax
from jax.experimental import pallas as pl
from jax.experimental.pallas import tpu as pltpu

_NUM_EMB = 1024
_DIM = 64
_TOP_K = 3
_COMMIT = 0.25


def _vq_body(x_ref, e_ref, dist_ref, idx_ref, q_ref, loss_ref, perp_ref,
             cnt_ref, sse_ref, *, nb, n_total, num_blocks):
    i = pl.program_id(0)
    x = x_ref[...]
    e = e_ref[...]
    x2 = jnp.sum(x * x, axis=1, keepdims=True)
    e2 = jnp.sum(e * e, axis=1, keepdims=True).T
    xe = jnp.dot(x, e.T, preferred_element_type=jnp.float32)
    d = x2 + e2 - 2.0 * xe
    dist_ref[...] = d

    iota = lax.broadcasted_iota(jnp.int32, (nb, _NUM_EMB), 1)
    work = d
    cnt = jnp.zeros((1, _NUM_EMB), jnp.float32)
    sse = jnp.zeros((), jnp.float32)
    idx_cols = []
    q_slabs = []
    for _ in range(_TOP_K):
        m = jnp.min(work, axis=1, keepdims=True)
        idxj = jnp.min(jnp.where(work == m, iota, _NUM_EMB),
                       axis=1, keepdims=True)
        hit = iota == idxj
        oh = hit.astype(jnp.float32)
        qj = jnp.dot(oh, e, preferred_element_type=jnp.float32)
        idx_cols.append(idxj)
        q_slabs.append(qj[:, None, :])
        cnt = cnt + jnp.sum(oh, axis=0, keepdims=True)
        diff = qj - x
        sse = sse + jnp.sum(diff * diff)
        work = jnp.where(hit, jnp.inf, work)

    idx_ref[...] = jnp.concatenate(idx_cols, axis=1)
    q_ref[...] = jnp.concatenate(q_slabs, axis=1)

    sse11 = sse.reshape(1, 1)

    @pl.when(i == 0)
    def _init():
        cnt_ref[...] = cnt
        sse_ref[...] = sse11
        loss_ref[...] = jnp.zeros((1, 1), jnp.float32)
        perp_ref[...] = jnp.zeros((1, 1), jnp.float32)

    @pl.when(i > 0)
    def _acc():
        cnt_ref[...] = cnt_ref[...] + cnt
        sse_ref[...] = sse_ref[...] + sse11

    @pl.when(i == num_blocks - 1)
    def _fin():
        avg = cnt_ref[...] / n_total
        ent = jnp.sum(avg * jnp.log(avg + 1e-10), keepdims=True)
        perp_ref[...] = jnp.exp(-ent).reshape(1, 1)
        loss_ref[...] = sse_ref[...] * (_COMMIT / (n_total * _TOP_K * _DIM))


def kernel(inputs, embedding_weight):
    B, T, C = inputs.shape
    N = B * T
    K = embedding_weight.shape[0]
    nb = 1024
    num_blocks = N // nb
    flat = inputs.reshape(N, C)

    import functools
    dist, idx, q, loss, perp = pl.pallas_call(
        functools.partial(_vq_body, nb=nb, n_total=float(N),
                          num_blocks=num_blocks),
        grid=(num_blocks,),
        in_specs=[
            pl.BlockSpec((nb, C), lambda i: (i, 0)),
            pl.BlockSpec((K, C), lambda i: (0, 0)),
        ],
        out_specs=[
            pl.BlockSpec((nb, K), lambda i: (i, 0)),
            pl.BlockSpec((nb, _TOP_K), lambda i: (i, 0)),
            pl.BlockSpec((nb, _TOP_K, C), lambda i: (i, 0, 0)),
            pl.BlockSpec((1, 1), lambda i: (0, 0)),
            pl.BlockSpec((1, 1), lambda i: (0, 0)),
        ],
        out_shape=[
            jax.ShapeDtypeStruct((N, K), jnp.float32),
            jax.ShapeDtypeStruct((N, _TOP_K), jnp.int32),
            jax.ShapeDtypeStruct((N, _TOP_K, C), jnp.float32),
            jax.ShapeDtypeStruct((1, 1), jnp.float32),
            jax.ShapeDtypeStruct((1, 1), jnp.float32),
        ],
        scratch_shapes=[
            pltpu.VMEM((1, K), jnp.float32),
            pltpu.VMEM((1, 1), jnp.float32),
        ],
    )(flat, embedding_weight)

    quantized_st = q.reshape(B, T, _TOP_K, C)
    return (loss[0, 0], quantized_st, perp[0, 0], idx, dist)
